# baseline (speedup 1.0000x reference)
import jax
import jax.numpy as jnp
from jax.experimental import pallas as pl
from jax.experimental.pallas import tpu as pltpu


def _expand_kernel(x_ref, w_ref, b_ref, o_ref, xb_ref):
    j = pl.program_id(1)

    @pl.when(j == 0)
    def _():
        xb_ref[...] = x_ref[...].astype(jnp.bfloat16)

    acc = jnp.dot(xb_ref[...], w_ref[...].astype(jnp.bfloat16),
                  preferred_element_type=jnp.float32)
    o_ref[...] = (acc + b_ref[...].astype(jnp.float32)).astype(o_ref.dtype)


def kernel(x, weight, bias):
    B, Cin = x.shape
    F = weight.shape[1]
    out_channels, Y, X = 128, 8, 8

    num_i = 2 if B % 2 == 0 else 1
    TM = B // num_i
    TN = next((t for t in (512, 256, 128) if F % t == 0), F)
    num_j = F // TN

    out_flat = pl.pallas_call(
        _expand_kernel,
        out_shape=jax.ShapeDtypeStruct((B, F), x.dtype),
        grid=(num_i, num_j),
        in_specs=[
            pl.BlockSpec((TM, Cin), lambda i, j: (i, 0)),
            pl.BlockSpec((Cin, TN), lambda i, j: (0, j)),
            pl.BlockSpec((1, TN), lambda i, j: (0, j)),
        ],
        out_specs=pl.BlockSpec((TM, TN), lambda i, j: (i, j)),
        scratch_shapes=[pltpu.VMEM((TM, Cin), jnp.bfloat16)],
        compiler_params=pltpu.CompilerParams(
            dimension_semantics=("parallel", "arbitrary")),
        cost_estimate=pl.CostEstimate(
            flops=2 * B * Cin * F,
            transcendentals=0,
            bytes_accessed=(B * Cin + Cin * F + B * F) * 4,
        ),
    )(x, weight, bias.reshape(1, F))

    return out_flat.reshape(B, out_channels, Y, X)
